# baseline (speedup 1.0000x reference)
import functools

import jax
import jax.numpy as jnp
from jax import lax
from jax.experimental import pallas as pl
from jax.experimental.pallas import tpu as pltpu
from jax.experimental.pallas import tpu_sc as plsc

N = 10000
DIN = 256
DOUT = 256
HE = 1024
E = 80000
BLK = 1000
NBLK = N // BLK
NC = 2
NS = 16
HALF = DOUT // 2
ROWS_PER_TILE = N // NS
EB = 128
NB_E = E // EB
NB_ALL = 2 * E // EB
NP = N + 16
EPT = 5120
EPAD = NS * EPT
AB = 80
NBT = EPT // AB
SROWS = 10112
ZCH = 64


def _dense_ab(nr_ref, wr_ref, protos_ref, theta_ref, cbias_ref, mpad_ref,
              h2_out, ap_out, x_cache, de_acc, g_acc, m_acc):
    p = pl.program_id(1)
    b = pl.program_id(2)

    @pl.when(p == 0)
    def _():
        x = jnp.dot(nr_ref[...], wr_ref[0], preferred_element_type=jnp.float32)
        hl = lax.dot_general(x, protos_ref[0], (((1,), (1,)), ((), ())),
                             preferred_element_type=jnp.float32)
        h = jax.nn.sigmoid(hl * (1.0 / 16.0))
        dv = jnp.sum(h, axis=1)
        dvi = jnp.where(dv > 0, lax.rsqrt(dv), 0.0)
        hn = h * dvi[:, None]

        @pl.when(b == 0)
        def _():
            de_acc[...] = jnp.zeros_like(de_acc)
            g_acc[...] = jnp.zeros_like(g_acc)

        de_acc[...] += jnp.sum(h, axis=0)
        g_acc[...] += lax.dot_general(hn, x, (((0,), (0,)), ((), ())),
                                      preferred_element_type=jnp.float32)
        x_cache[pl.ds(b * BLK, BLK), :] = x

    @pl.when(p == 1)
    def _():
        x = x_cache[pl.ds(b * BLK, BLK), :]
        hl = lax.dot_general(x, protos_ref[0], (((1,), (1,)), ((), ())),
                             preferred_element_type=jnp.float32)
        h = jax.nn.sigmoid(hl * (1.0 / 16.0))
        dv = jnp.sum(h, axis=1)
        dvi = jnp.where(dv > 0, lax.rsqrt(dv), 0.0)
        hn = h * dvi[:, None]

        @pl.when(b == 0)
        def _():
            de = de_acc[...]
            dei = jnp.where(de > 0, 1.0 / de, 0.0)
            m_acc[...] = dei[:, None] * g_acc[...]

        agg = jnp.dot(hn, m_acc[...], preferred_element_type=jnp.float32)
        out = (jnp.dot(agg, theta_ref[0], preferred_element_type=jnp.float32)
               + cbias_ref[0])
        h2_out[0, 0] = out[:, :HALF]
        h2_out[0, 1] = out[:, HALF:]
        ap_out[0] = jnp.dot(out, mpad_ref[...],
                            preferred_element_type=jnp.float32)


def _final_c(hp_ref, hn_ref, bias_ref, out_ref):
    out_ref[...] = hp_ref[...] - hn_ref[...] + bias_ref[...]


_SC_MESH = plsc.VectorSubcoreMesh(core_axis_name="c", subcore_axis_name="s",
                                  num_cores=NC, num_subcores=NS)


@functools.partial(
    pl.kernel,
    out_type=jax.ShapeDtypeStruct((NC, N, 16), jnp.float32),
    mesh=_SC_MESH,
    compiler_params=pltpu.CompilerParams(use_tc_tiling_on_sc=False, needs_layout_passes=False),
    scratch_types=[
        pltpu.VMEM((EB,), jnp.int32),
        pltpu.VMEM((EB, 16), jnp.float32),
        pltpu.VMEM((EB, 16), jnp.float32),
        pltpu.VMEM_SHARED((N, 16), jnp.float32),
    ],
)
def _deg_kernel(src_hbm, out_hbm, idx_v, ones_v, zer_v, shared):
    cid = lax.axis_index("c")
    sid = lax.axis_index("s")
    wid = cid * NS + sid

    def fill(i, carry):
        ones_v[i] = jnp.full((16,), 1.0, jnp.float32)
        zer_v[i] = jnp.zeros((16,), jnp.float32)
        return carry

    lax.fori_loop(0, EB, fill, 0)
    for ch in range(5):
        pltpu.sync_copy(zer_v.at[pl.ds(0, 125)],
                        shared.at[pl.ds(sid * ROWS_PER_TILE + ch * 125, 125)])
    plsc.subcore_barrier()

    base_nb = NB_ALL // (NC * NS)
    extra = NB_ALL - base_nb * NC * NS
    nb = base_nb + jnp.where(wid < extra, 1, 0)

    def body(i, carry):
        base = (wid + i * NC * NS) * EB
        pltpu.sync_copy(src_hbm.at[pl.ds(base, EB)], idx_v)
        pltpu.sync_copy(ones_v, shared.at[idx_v], add=True)
        return carry

    lax.fori_loop(0, nb, body, 0)
    plsc.subcore_barrier()
    pltpu.sync_copy(shared.at[pl.ds(sid * ROWS_PER_TILE, ROWS_PER_TILE)],
                    out_hbm.at[cid, pl.ds(sid * ROWS_PER_TILE, ROWS_PER_TILE)])


@functools.partial(
    pl.kernel,
    out_type=jax.ShapeDtypeStruct((2, EPAD), jnp.float32),
    mesh=_SC_MESH,
    compiler_params=pltpu.CompilerParams(use_tc_tiling_on_sc=False, needs_layout_passes=False),
    scratch_types=[
        pltpu.VMEM((3, NP), jnp.float32),
        pltpu.VMEM((EPT,), jnp.int32),
        pltpu.VMEM((EPT,), jnp.int32),
        pltpu.VMEM((EPT,), jnp.float32),
    ],
)
def _coeff_kernel(adj_hbm, tabs_hbm, out_hbm, tabs_v, srcs_v, dsts_v, cf_v):
    sign = lax.axis_index("c")
    sid = lax.axis_index("s")
    ebase = sid * EPT
    pltpu.sync_copy(tabs_hbm.at[sign], tabs_v)
    pltpu.sync_copy(adj_hbm.at[sign, 0, pl.ds(ebase, EPT)], srcs_v)
    pltpu.sync_copy(adj_hbm.at[sign, 1, pl.ds(ebase, EPT)], dsts_v)
    asrc_row = jnp.full((16,), 0, jnp.int32)
    adst_row = jnp.full((16,), 1, jnp.int32)
    isq_row = jnp.full((16,), 2, jnp.int32)

    def body(g, carry):
        sl = pl.ds(16 * g, 16)
        s16 = srcs_v[sl]
        d16 = dsts_v[sl]
        a = (plsc.load_gather(tabs_v, [asrc_row, s16])
             + plsc.load_gather(tabs_v, [adst_row, d16]))
        lr = jnp.maximum(a, 0.2 * a)
        sg = 1.0 / (1.0 + jnp.exp(-lr))
        cf_v[sl] = (sg * plsc.load_gather(tabs_v, [isq_row, s16])
                    * plsc.load_gather(tabs_v, [isq_row, d16]))
        return carry

    lax.fori_loop(0, EPT // 16, body, 0)
    pltpu.sync_copy(cf_v, out_hbm.at[sign, pl.ds(ebase, EPT)])


@functools.partial(
    pl.kernel,
    out_type=jax.ShapeDtypeStruct((2, NC, N, HALF), jnp.float32),
    mesh=_SC_MESH,
    compiler_params=pltpu.CompilerParams(use_tc_tiling_on_sc=False, needs_layout_passes=False),
    scratch_types=[
        pltpu.VMEM((EPT,), jnp.float32),
        pltpu.VMEM((EPT,), jnp.int32),
        pltpu.VMEM((EPT,), jnp.int32),
        pltpu.VMEM((AB,), jnp.int32),
        pltpu.VMEM((AB,), jnp.int32),
        pltpu.VMEM((AB,), jnp.int32),
        pltpu.VMEM((AB,), jnp.int32),
        pltpu.VMEM((AB,), jnp.int32),
        pltpu.VMEM((AB,), jnp.int32),
        pltpu.VMEM((AB, HALF), jnp.float32),
        pltpu.VMEM((AB, HALF), jnp.float32),
        pltpu.VMEM((AB, HALF), jnp.float32),
        pltpu.VMEM_SHARED((SROWS, HALF), jnp.float32),
        pltpu.SemaphoreType.DMA,
        pltpu.SemaphoreType.DMA,
        pltpu.SemaphoreType.DMA,
        pltpu.SemaphoreType.DMA,
        pltpu.SemaphoreType.DMA,
        pltpu.SemaphoreType.DMA,
    ],
)
def _agg_kernel(h2_hbm, adj_hbm, cf_hbm, out_hbm, cfs_v, srcs_v, dsts_v,
                gix0, gix1, gix2, six0, six1, six2, rows0, rows1, rows2,
                shared, gsem0, gsem1, gsem2, ssem0, ssem1, ssem2):
    cid = lax.axis_index("c")
    sid = lax.axis_index("s")
    gixs = (gix0, gix1, gix2)
    sixs = (six0, six1, six2)
    rowss = (rows0, rows1, rows2)
    gsems = (gsem0, gsem1, gsem2)
    ssems = (ssem0, ssem1, ssem2)

    def zf(i, carry):
        for j in range(HALF // 16):
            rows0[i, pl.ds(16 * j, 16)] = jnp.zeros((16,), jnp.float32)
        return carry

    def zero_shared():
        lax.fori_loop(0, ZCH, zf, 0)
        for i in range(10):
            ch = sid + NS * i

            @pl.when(ch < SROWS // ZCH)
            def _():
                pltpu.sync_copy(rows0.at[pl.ds(0, ZCH)],
                                shared.at[pl.ds(ch * ZCH, ZCH)])

    zero_shared()
    plsc.subcore_barrier()

    for sign in range(2):
        half_off = (sign * 2 + cid) * N
        ebase = sid * EPT
        pltpu.sync_copy(adj_hbm.at[sign, 0, pl.ds(ebase, EPT)], srcs_v)
        pltpu.sync_copy(adj_hbm.at[sign, 1, pl.ds(ebase, EPT)], dsts_v)
        pltpu.sync_copy(cf_hbm.at[sign, pl.ds(ebase, EPT)], cfs_v)

        def prep(j, b):
            @pl.when(j >= 3)
            def _():
                pltpu.make_async_copy(rowss[b], shared.at[sixs[b]],
                                      ssems[b]).wait()
            for k in range(AB // 16):
                sl = pl.ds(j * AB + 16 * k, 16)
                o = pl.ds(16 * k, 16)
                sixs[b][o] = srcs_v[sl]
                gixs[b][o] = dsts_v[sl] + half_off
            pltpu.async_copy(h2_hbm.at[gixs[b]], rowss[b], gsems[b])

        def finish(j, b):
            pltpu.make_async_copy(h2_hbm.at[gixs[b]], rowss[b], gsems[b]).wait()
            cbase = j * AB

            def scale(g, c2):
                c16 = cfs_v[pl.ds(cbase + 16 * g, 16)]
                for l in range(16):
                    cb = jnp.broadcast_to(c16[l], (16,))
                    e = 16 * g + l
                    for jj in range(HALF // 16):
                        slf = pl.ds(16 * jj, 16)
                        rowss[b][e, slf] = rowss[b][e, slf] * cb
                return c2

            lax.fori_loop(0, AB // 16, scale, 0)
            pltpu.async_copy(rowss[b], shared.at[sixs[b]], ssems[b], add=True)

        def ring(p, carry):
            for q in range(3):
                j = 3 * p + q

                @pl.when(j < NBT)
                def _():
                    prep(j, q)

                @pl.when(j >= 2)
                def _():
                    finish(j - 2, (q + 1) % 3)
            return carry

        lax.fori_loop(0, (NBT + 2) // 3, ring, 0)
        for b in range(3):
            pltpu.make_async_copy(rowss[b], shared.at[sixs[b]],
                                  ssems[b]).wait()
        plsc.subcore_barrier()
        pltpu.sync_copy(shared.at[pl.ds(sid * ROWS_PER_TILE, ROWS_PER_TILE)],
                        out_hbm.at[sign, cid,
                                   pl.ds(sid * ROWS_PER_TILE, ROWS_PER_TILE)])
        if sign == 0:
            plsc.subcore_barrier()
            zero_shared()
            plsc.subcore_barrier()


def kernel(node_reps, adj_pos, adj_neg, basis, att, bias, mapping_func,
           theta1, bias1, protos1, theta2, bias2, protos2):
    f32 = jnp.float32
    Wr = (att @ basis.reshape(2, -1)).reshape(2, DIN, DOUT)
    protos = jnp.stack([protos1, protos2])
    theta = jnp.stack([theta1, theta2])
    cbias = jnp.stack([bias1, bias2])[:, None, :]
    m1 = mapping_func[0, :DOUT]
    m2 = mapping_func[0, DOUT:]
    mpad = jnp.zeros((DOUT, 128), f32).at[:, 0].set(m1).at[:, 1].set(m2)

    h2_all, ap_all = pl.pallas_call(
        _dense_ab,
        grid=(2, 2, NBLK),
        in_specs=[
            pl.BlockSpec((BLK, DIN), lambda s, p, b: (b, 0)),
            pl.BlockSpec((1, DIN, DOUT), lambda s, p, b: (s, 0, 0)),
            pl.BlockSpec((1, HE, DOUT), lambda s, p, b: (s, 0, 0)),
            pl.BlockSpec((1, DOUT, DOUT), lambda s, p, b: (s, 0, 0)),
            pl.BlockSpec((1, 1, DOUT), lambda s, p, b: (s, 0, 0)),
            pl.BlockSpec((DOUT, 128), lambda s, p, b: (0, 0)),
        ],
        out_specs=[
            pl.BlockSpec((1, 2, BLK, HALF), lambda s, p, b: (s, 0, b, 0)),
            pl.BlockSpec((1, BLK, 128), lambda s, p, b: (s, b, 0)),
        ],
        out_shape=[
            jax.ShapeDtypeStruct((2, 2, N, HALF), f32),
            jax.ShapeDtypeStruct((2, N, 128), f32),
        ],
        scratch_shapes=[pltpu.VMEM((N, DIN), f32), pltpu.VMEM((HE,), f32),
                        pltpu.VMEM((HE, DOUT), f32), pltpu.VMEM((HE, DOUT), f32)],
    )(node_reps, Wr, protos, theta, cbias, mpad)

    src_all = jnp.concatenate([adj_pos[0], adj_neg[0]])
    deg_parts = _deg_kernel(src_all)
    deg = deg_parts[0, :, 0] + deg_parts[1, :, 0]
    isq = lax.rsqrt(deg)

    npad = EPAD - E
    spad = jnp.full((npad,), N, jnp.int32)
    dpad = jnp.zeros((npad,), jnp.int32)
    adj_pad = jnp.stack([
        jnp.stack([jnp.concatenate([adj_pos[0], spad]),
                   jnp.concatenate([adj_pos[1], dpad])]),
        jnp.stack([jnp.concatenate([adj_neg[0], spad]),
                   jnp.concatenate([adj_neg[1], dpad])]),
    ])
    tpad = jnp.zeros((2, 3, NP - N), f32)
    tabs = jnp.concatenate([
        jnp.stack([jnp.stack([ap_all[0, :, 0], ap_all[0, :, 1], isq]),
                   jnp.stack([ap_all[1, :, 0], ap_all[1, :, 1], isq])]),
        tpad], axis=2)
    cf = _coeff_kernel(adj_pad, tabs)
    h2 = h2_all.reshape(4 * N, HALF)
    out_agg = _agg_kernel(h2, adj_pad, cf)
    h_agg_pos = jnp.concatenate([out_agg[0, 0], out_agg[0, 1]], axis=1)
    h_agg_neg = jnp.concatenate([out_agg[1, 0], out_agg[1, 1]], axis=1)

    output = pl.pallas_call(
        _final_c,
        grid=(NBLK,),
        in_specs=[
            pl.BlockSpec((BLK, DOUT), lambda b: (b, 0)),
            pl.BlockSpec((BLK, DOUT), lambda b: (b, 0)),
            pl.BlockSpec((1, DOUT), lambda b: (0, 0)),
        ],
        out_specs=pl.BlockSpec((BLK, DOUT), lambda b: (b, 0)),
        out_shape=jax.ShapeDtypeStruct((N, DOUT), f32),
    )(h_agg_pos, h_agg_neg, bias)

    return (output, h_agg_pos, h_agg_neg)


# speedup vs baseline: 1.0276x; 1.0276x over previous
import functools

import jax
import jax.numpy as jnp
from jax import lax
from jax.experimental import pallas as pl
from jax.experimental.pallas import tpu as pltpu
from jax.experimental.pallas import tpu_sc as plsc

N = 10000
DIN = 256
DOUT = 256
HE = 1024
E = 80000
BLK = 1000
NBLK = N // BLK
NC = 2
NS = 16
HALF = DOUT // 2
ROWS_PER_TILE = N // NS
EB = 128
NB_E = E // EB
NB_ALL = 2 * E // EB
NP = N + 16
EPT = 5120
EPAD = NS * EPT
AB = 80
NBT = EPT // AB
SROWS = 10112
ZCH = 64


def _dense_a(nr_ref, wr_ref, protos_ref, x_out, de_out, g_out, de_acc, g_acc):
    b = pl.program_id(1)
    x = jnp.dot(nr_ref[...], wr_ref[0], preferred_element_type=jnp.float32)
    hl = lax.dot_general(x, protos_ref[0], (((1,), (1,)), ((), ())),
                         preferred_element_type=jnp.float32)
    h = jax.nn.sigmoid(hl * (1.0 / 16.0))
    dv = jnp.sum(h, axis=1)
    dvi = jnp.where(dv > 0, lax.rsqrt(dv), 0.0)
    hn = h * dvi[:, None]

    @pl.when(b == 0)
    def _():
        de_acc[...] = jnp.zeros_like(de_acc)
        g_acc[...] = jnp.zeros_like(g_acc)

    de_acc[...] += jnp.sum(h, axis=0)
    g_acc[...] += lax.dot_general(hn, x, (((0,), (0,)), ((), ())),
                                  preferred_element_type=jnp.float32)
    x_out[...] = x[None]

    @pl.when(b == NBLK - 1)
    def _():
        de_out[...] = de_acc[...][None, None]
        g_out[...] = g_acc[...][None]


def _dense_b(x_ref, protos_ref, de_ref, g_ref, theta_ref, cbias_ref, mpad_ref,
             h2_out, ap_out, m_acc):
    b = pl.program_id(1)
    x = x_ref[0]
    hl = lax.dot_general(x, protos_ref[0], (((1,), (1,)), ((), ())),
                         preferred_element_type=jnp.float32)
    h = jax.nn.sigmoid(hl * (1.0 / 16.0))
    dv = jnp.sum(h, axis=1)
    dvi = jnp.where(dv > 0, lax.rsqrt(dv), 0.0)
    hn = h * dvi[:, None]

    @pl.when(b == 0)
    def _():
        de = de_ref[0, 0]
        dei = jnp.where(de > 0, 1.0 / de, 0.0)
        m_acc[...] = dei[:, None] * g_ref[0]

    agg = jnp.dot(hn, m_acc[...], preferred_element_type=jnp.float32)
    out = jnp.dot(agg, theta_ref[0], preferred_element_type=jnp.float32) + cbias_ref[0]
    h2_out[0, 0] = out[:, :HALF]
    h2_out[0, 1] = out[:, HALF:]
    ap_out[0] = jnp.dot(out, mpad_ref[...], preferred_element_type=jnp.float32)


def _final_c(hp_ref, hn_ref, bias_ref, out_ref):
    out_ref[...] = hp_ref[...] - hn_ref[...] + bias_ref[...]


_SC_MESH = plsc.VectorSubcoreMesh(core_axis_name="c", subcore_axis_name="s",
                                  num_cores=NC, num_subcores=NS)


@functools.partial(
    pl.kernel,
    out_type=jax.ShapeDtypeStruct((NC, N, 16), jnp.float32),
    mesh=_SC_MESH,
    compiler_params=pltpu.CompilerParams(use_tc_tiling_on_sc=False, needs_layout_passes=False),
    scratch_types=[
        pltpu.VMEM((EB,), jnp.int32),
        pltpu.VMEM((EB, 16), jnp.float32),
        pltpu.VMEM((EB, 16), jnp.float32),
        pltpu.VMEM_SHARED((N, 16), jnp.float32),
    ],
)
def _deg_kernel(src_hbm, out_hbm, idx_v, ones_v, zer_v, shared):
    cid = lax.axis_index("c")
    sid = lax.axis_index("s")
    wid = cid * NS + sid

    def fill(i, carry):
        ones_v[i] = jnp.full((16,), 1.0, jnp.float32)
        zer_v[i] = jnp.zeros((16,), jnp.float32)
        return carry

    lax.fori_loop(0, EB, fill, 0)
    for ch in range(5):
        pltpu.sync_copy(zer_v.at[pl.ds(0, 125)],
                        shared.at[pl.ds(sid * ROWS_PER_TILE + ch * 125, 125)])
    plsc.subcore_barrier()

    base_nb = NB_ALL // (NC * NS)
    extra = NB_ALL - base_nb * NC * NS
    nb = base_nb + jnp.where(wid < extra, 1, 0)

    def body(i, carry):
        base = (wid + i * NC * NS) * EB
        pltpu.sync_copy(src_hbm.at[pl.ds(base, EB)], idx_v)
        pltpu.sync_copy(ones_v, shared.at[idx_v], add=True)
        return carry

    lax.fori_loop(0, nb, body, 0)
    plsc.subcore_barrier()
    pltpu.sync_copy(shared.at[pl.ds(sid * ROWS_PER_TILE, ROWS_PER_TILE)],
                    out_hbm.at[cid, pl.ds(sid * ROWS_PER_TILE, ROWS_PER_TILE)])


@functools.partial(
    pl.kernel,
    out_type=jax.ShapeDtypeStruct((2, EPAD), jnp.float32),
    mesh=_SC_MESH,
    compiler_params=pltpu.CompilerParams(use_tc_tiling_on_sc=False, needs_layout_passes=False),
    scratch_types=[
        pltpu.VMEM((3, NP), jnp.float32),
        pltpu.VMEM((EPT,), jnp.int32),
        pltpu.VMEM((EPT,), jnp.int32),
        pltpu.VMEM((EPT,), jnp.float32),
    ],
)
def _coeff_kernel(adj_hbm, tabs_hbm, out_hbm, tabs_v, srcs_v, dsts_v, cf_v):
    sign = lax.axis_index("c")
    sid = lax.axis_index("s")
    ebase = sid * EPT
    pltpu.sync_copy(tabs_hbm.at[sign], tabs_v)
    pltpu.sync_copy(adj_hbm.at[sign, 0, pl.ds(ebase, EPT)], srcs_v)
    pltpu.sync_copy(adj_hbm.at[sign, 1, pl.ds(ebase, EPT)], dsts_v)
    asrc_row = jnp.full((16,), 0, jnp.int32)
    adst_row = jnp.full((16,), 1, jnp.int32)
    isq_row = jnp.full((16,), 2, jnp.int32)

    def body(g, carry):
        sl = pl.ds(16 * g, 16)
        s16 = srcs_v[sl]
        d16 = dsts_v[sl]
        a = (plsc.load_gather(tabs_v, [asrc_row, s16])
             + plsc.load_gather(tabs_v, [adst_row, d16]))
        lr = jnp.maximum(a, 0.2 * a)
        sg = 1.0 / (1.0 + jnp.exp(-lr))
        cf_v[sl] = (sg * plsc.load_gather(tabs_v, [isq_row, s16])
                    * plsc.load_gather(tabs_v, [isq_row, d16]))
        return carry

    lax.fori_loop(0, EPT // 16, body, 0)
    pltpu.sync_copy(cf_v, out_hbm.at[sign, pl.ds(ebase, EPT)])


@functools.partial(
    pl.kernel,
    out_type=jax.ShapeDtypeStruct((2, NC, N, HALF), jnp.float32),
    mesh=_SC_MESH,
    compiler_params=pltpu.CompilerParams(use_tc_tiling_on_sc=False, needs_layout_passes=False),
    scratch_types=[
        pltpu.VMEM((EPT,), jnp.float32),
        pltpu.VMEM((EPT,), jnp.int32),
        pltpu.VMEM((EPT,), jnp.int32),
        pltpu.VMEM((AB,), jnp.int32),
        pltpu.VMEM((AB,), jnp.int32),
        pltpu.VMEM((AB,), jnp.int32),
        pltpu.VMEM((AB,), jnp.int32),
        pltpu.VMEM((AB,), jnp.int32),
        pltpu.VMEM((AB,), jnp.int32),
        pltpu.VMEM((AB, HALF), jnp.float32),
        pltpu.VMEM((AB, HALF), jnp.float32),
        pltpu.VMEM((AB, HALF), jnp.float32),
        pltpu.VMEM_SHARED((SROWS, HALF), jnp.float32),
        pltpu.SemaphoreType.DMA,
        pltpu.SemaphoreType.DMA,
        pltpu.SemaphoreType.DMA,
        pltpu.SemaphoreType.DMA,
        pltpu.SemaphoreType.DMA,
        pltpu.SemaphoreType.DMA,
    ],
)
def _agg_kernel(h2_hbm, adj_hbm, cf_hbm, out_hbm, cfs_v, srcs_v, dsts_v,
                gix0, gix1, gix2, six0, six1, six2, rows0, rows1, rows2,
                shared, gsem0, gsem1, gsem2, ssem0, ssem1, ssem2):
    cid = lax.axis_index("c")
    sid = lax.axis_index("s")
    gixs = (gix0, gix1, gix2)
    sixs = (six0, six1, six2)
    rowss = (rows0, rows1, rows2)
    gsems = (gsem0, gsem1, gsem2)
    ssems = (ssem0, ssem1, ssem2)

    def zf(i, carry):
        for j in range(HALF // 16):
            rows0[i, pl.ds(16 * j, 16)] = jnp.zeros((16,), jnp.float32)
        return carry

    def zero_shared():
        lax.fori_loop(0, ZCH, zf, 0)
        for i in range(10):
            ch = sid + NS * i

            @pl.when(ch < SROWS // ZCH)
            def _():
                pltpu.sync_copy(rows0.at[pl.ds(0, ZCH)],
                                shared.at[pl.ds(ch * ZCH, ZCH)])

    zero_shared()
    plsc.subcore_barrier()

    for sign in range(2):
        half_off = (sign * 2 + cid) * N
        ebase = sid * EPT
        pltpu.sync_copy(adj_hbm.at[sign, 0, pl.ds(ebase, EPT)], srcs_v)
        pltpu.sync_copy(adj_hbm.at[sign, 1, pl.ds(ebase, EPT)], dsts_v)
        pltpu.sync_copy(cf_hbm.at[sign, pl.ds(ebase, EPT)], cfs_v)

        def prep(j, b):
            @pl.when(j >= 3)
            def _():
                pltpu.make_async_copy(rowss[b], shared.at[sixs[b]],
                                      ssems[b]).wait()
            for k in range(AB // 16):
                sl = pl.ds(j * AB + 16 * k, 16)
                o = pl.ds(16 * k, 16)
                sixs[b][o] = srcs_v[sl]
                gixs[b][o] = dsts_v[sl] + half_off
            pltpu.async_copy(h2_hbm.at[gixs[b]], rowss[b], gsems[b])

        def finish(j, b):
            pltpu.make_async_copy(h2_hbm.at[gixs[b]], rowss[b], gsems[b]).wait()
            cbase = j * AB

            def scale(g, c2):
                c16 = cfs_v[pl.ds(cbase + 16 * g, 16)]
                for l in range(16):
                    cb = jnp.broadcast_to(c16[l], (16,))
                    e = 16 * g + l
                    for jj in range(HALF // 16):
                        slf = pl.ds(16 * jj, 16)
                        rowss[b][e, slf] = rowss[b][e, slf] * cb
                return c2

            lax.fori_loop(0, AB // 16, scale, 0)
            pltpu.async_copy(rowss[b], shared.at[sixs[b]], ssems[b], add=True)

        def ring(p, carry):
            for q in range(3):
                j = 3 * p + q

                @pl.when(j < NBT)
                def _():
                    prep(j, q)

                @pl.when(j >= 2)
                def _():
                    finish(j - 2, (q + 1) % 3)
            return carry

        lax.fori_loop(0, (NBT + 2) // 3, ring, 0)
        for b in range(3):
            pltpu.make_async_copy(rowss[b], shared.at[sixs[b]],
                                  ssems[b]).wait()
        plsc.subcore_barrier()
        pltpu.sync_copy(shared.at[pl.ds(sid * ROWS_PER_TILE, ROWS_PER_TILE)],
                        out_hbm.at[sign, cid,
                                   pl.ds(sid * ROWS_PER_TILE, ROWS_PER_TILE)])
        if sign == 0:
            plsc.subcore_barrier()
            zero_shared()
            plsc.subcore_barrier()


def kernel(node_reps, adj_pos, adj_neg, basis, att, bias, mapping_func,
           theta1, bias1, protos1, theta2, bias2, protos2):
    f32 = jnp.float32
    Wr = (att @ basis.reshape(2, -1)).reshape(2, DIN, DOUT)
    protos = jnp.stack([protos1, protos2])
    theta = jnp.stack([theta1, theta2])
    cbias = jnp.stack([bias1, bias2])[:, None, :]
    m1 = mapping_func[0, :DOUT]
    m2 = mapping_func[0, DOUT:]
    mpad = jnp.zeros((DOUT, 128), f32).at[:, 0].set(m1).at[:, 1].set(m2)

    x_all, de, g = pl.pallas_call(
        _dense_a,
        grid=(2, NBLK),
        in_specs=[
            pl.BlockSpec((BLK, DIN), lambda s, b: (b, 0)),
            pl.BlockSpec((1, DIN, DOUT), lambda s, b: (s, 0, 0)),
            pl.BlockSpec((1, HE, DOUT), lambda s, b: (s, 0, 0)),
        ],
        out_specs=[
            pl.BlockSpec((1, BLK, DOUT), lambda s, b: (s, b, 0)),
            pl.BlockSpec((1, 1, HE), lambda s, b: (s, 0, 0)),
            pl.BlockSpec((1, HE, DOUT), lambda s, b: (s, 0, 0)),
        ],
        out_shape=[
            jax.ShapeDtypeStruct((2, N, DOUT), f32),
            jax.ShapeDtypeStruct((2, 1, HE), f32),
            jax.ShapeDtypeStruct((2, HE, DOUT), f32),
        ],
        scratch_shapes=[pltpu.VMEM((HE,), f32), pltpu.VMEM((HE, DOUT), f32)],
    )(node_reps, Wr, protos)

    h2_all, ap_all = pl.pallas_call(
        _dense_b,
        grid=(2, NBLK),
        in_specs=[
            pl.BlockSpec((1, BLK, DIN), lambda s, b: (s, b, 0)),
            pl.BlockSpec((1, HE, DOUT), lambda s, b: (s, 0, 0)),
            pl.BlockSpec((1, 1, HE), lambda s, b: (s, 0, 0)),
            pl.BlockSpec((1, HE, DOUT), lambda s, b: (s, 0, 0)),
            pl.BlockSpec((1, DOUT, DOUT), lambda s, b: (s, 0, 0)),
            pl.BlockSpec((1, 1, DOUT), lambda s, b: (s, 0, 0)),
            pl.BlockSpec((DOUT, 128), lambda s, b: (0, 0)),
        ],
        out_specs=[
            pl.BlockSpec((1, 2, BLK, HALF), lambda s, b: (s, 0, b, 0)),
            pl.BlockSpec((1, BLK, 128), lambda s, b: (s, b, 0)),
        ],
        out_shape=[
            jax.ShapeDtypeStruct((2, 2, N, HALF), f32),
            jax.ShapeDtypeStruct((2, N, 128), f32),
        ],
        scratch_shapes=[pltpu.VMEM((HE, DOUT), f32)],
    )(x_all, protos, de, g, theta, cbias, mpad)

    src_all = jnp.concatenate([adj_pos[0], adj_neg[0]])
    deg_parts = _deg_kernel(src_all)
    deg = deg_parts[0, :, 0] + deg_parts[1, :, 0]
    isq = lax.rsqrt(deg)

    npad = EPAD - E
    spad = jnp.full((npad,), N, jnp.int32)
    dpad = jnp.zeros((npad,), jnp.int32)
    adj_pad = jnp.stack([
        jnp.stack([jnp.concatenate([adj_pos[0], spad]),
                   jnp.concatenate([adj_pos[1], dpad])]),
        jnp.stack([jnp.concatenate([adj_neg[0], spad]),
                   jnp.concatenate([adj_neg[1], dpad])]),
    ])
    tpad = jnp.zeros((2, 3, NP - N), f32)
    tabs = jnp.concatenate([
        jnp.stack([jnp.stack([ap_all[0, :, 0], ap_all[0, :, 1], isq]),
                   jnp.stack([ap_all[1, :, 0], ap_all[1, :, 1], isq])]),
        tpad], axis=2)
    cf = _coeff_kernel(adj_pad, tabs)
    h2 = h2_all.reshape(4 * N, HALF)
    out_agg = _agg_kernel(h2, adj_pad, cf)
    h_agg_pos = jnp.concatenate([out_agg[0, 0], out_agg[0, 1]], axis=1)
    h_agg_neg = jnp.concatenate([out_agg[1, 0], out_agg[1, 1]], axis=1)

    output = pl.pallas_call(
        _final_c,
        grid=(NBLK,),
        in_specs=[
            pl.BlockSpec((BLK, DOUT), lambda b: (b, 0)),
            pl.BlockSpec((BLK, DOUT), lambda b: (b, 0)),
            pl.BlockSpec((1, DOUT), lambda b: (0, 0)),
        ],
        out_specs=pl.BlockSpec((BLK, DOUT), lambda b: (b, 0)),
        out_shape=jax.ShapeDtypeStruct((N, DOUT), f32),
    )(h_agg_pos, h_agg_neg, bias)

    return (output, h_agg_pos, h_agg_neg)
